# baseline (speedup 1.0000x reference)
import functools

import jax
import jax.numpy as jnp
from jax import lax
from jax.experimental import pallas as pl
from jax.experimental.pallas import tpu as pltpu
from jax.experimental.pallas import tpu_sc as plsc

_N = 10000
_NP = 10240
_E = 640000
_CH = 56
_EPA = 642432
_FH = 160
_NBLK = 1024
_GRID = _NP // _NBLK

_mesh = plsc.VectorSubcoreMesh(core_axis_name="c", subcore_axis_name="s")
_sc_params = pltpu.CompilerParams(needs_layout_passes=False,
                                  use_tc_tiling_on_sc=False)


_DCH = 128
_EPD = 647168


@functools.partial(
    pl.kernel,
    out_type=jax.ShapeDtypeStruct((2 * _NP,), jnp.float32),
    mesh=_mesh,
    scratch_types=[
        pltpu.VMEM((2, _DCH), jnp.int32),
        pltpu.VMEM((2, _DCH), jnp.float32),
        pltpu.VMEM((_NP // 16,), jnp.float32),
        pltpu.SemaphoreType.DMA((2,)),
        pltpu.SemaphoreType.DMA((2,)),
        pltpu.VMEM_SHARED((_NP,), jnp.float32),
    ],
    compiler_params=_sc_params,
)
def _deg_kernel(row_hbm, ewm_hbm, deg_out,
                row_v, upd_v, z_v, slm, sd, deg_sh):
    c = lax.axis_index("c")
    s = lax.axis_index("s")
    nslice = _NP // 16

    def zb(j, _):
        z_v[pl.ds(j * 16, 16)] = jnp.zeros((16,), jnp.float32)
        return 0
    lax.fori_loop(0, nslice // 16, zb, 0)
    pltpu.sync_copy(z_v, deg_sh.at[pl.ds(s * nslice, nslice)])
    plsc.subcore_barrier()

    per_core = _EPD // 2
    per_tile = per_core // 16
    nch = per_tile // _DCH

    def issue_load(idx, b):
        base = pl.multiple_of(c * per_core + s * per_tile + idx * _DCH, 8)
        pltpu.async_copy(row_hbm.at[pl.ds(base, _DCH)], row_v.at[b],
                         slm.at[b])
        pltpu.async_copy(ewm_hbm.at[pl.ds(base, _DCH)], upd_v.at[b],
                         slm.at[b])

    def wait_load(idx, b):
        base = pl.multiple_of(c * per_core + s * per_tile + idx * _DCH, 8)
        pltpu.make_async_copy(row_hbm.at[pl.ds(base, _DCH)], row_v.at[b],
                              slm.at[b]).wait()
        pltpu.make_async_copy(ewm_hbm.at[pl.ds(base, _DCH)], upd_v.at[b],
                              slm.at[b]).wait()

    def scatter(b):
        pltpu.async_copy(upd_v.at[b], deg_sh.at[row_v.at[b]], sd.at[b],
                         add=True)

    def wait_scatter(b):
        pltpu.make_async_copy(upd_v.at[b], deg_sh.at[row_v.at[b]],
                              sd.at[b]).wait()

    issue_load(0, 0)

    def body(g, _):
        i0 = g * 2

        @pl.when(g >= 1)
        def _():
            wait_scatter(1)
        issue_load(i0 + 1, 1)
        wait_load(i0, 0)
        scatter(0)

        @pl.when(i0 + 2 < nch)
        def _():
            wait_scatter(0)
            issue_load(i0 + 2, 0)
        wait_load(i0 + 1, 1)
        scatter(1)
        return 0
    lax.fori_loop(0, nch // 2, body, 0)
    wait_scatter(0)
    wait_scatter(1)
    plsc.subcore_barrier()
    pltpu.sync_copy(deg_sh.at[pl.ds(s * nslice, nslice)],
                    deg_out.at[pl.ds(c * _NP + s * nslice, nslice)])


def _tc0_body(r_ref, c_ref, w_ref, o_ref):
    o_ref[...] = jnp.where(r_ref[...] == c_ref[...],
                           jnp.zeros_like(w_ref[...]), w_ref[...])


_tc0_call = pl.pallas_call(
    _tc0_body,
    grid=(5,),
    in_specs=[
        pl.BlockSpec((1000, 128), lambda i: (i, 0)),
        pl.BlockSpec((1000, 128), lambda i: (i, 0)),
        pl.BlockSpec((1000, 128), lambda i: (i, 0)),
    ],
    out_specs=pl.BlockSpec((1000, 128), lambda i: (i, 0)),
    out_shape=jax.ShapeDtypeStruct((5000, 128), jnp.float32),
)


@functools.partial(
    pl.kernel,
    out_type=jax.ShapeDtypeStruct((2 * _NP, _FH), jnp.float32),
    mesh=_mesh,
    scratch_types=[
        pltpu.VMEM((3, _CH), jnp.int32),
        pltpu.VMEM((3 * _CH,), jnp.int32),
        pltpu.VMEM((3 * _CH,), jnp.float32),
        pltpu.VMEM((3, _CH, _FH), jnp.float32),
        pltpu.SemaphoreType.DMA((3,)),
        pltpu.SemaphoreType.DMA((3,)),
        pltpu.VMEM_SHARED((_NP, _FH), jnp.float32),
    ],
    compiler_params=_sc_params,
)
def _agg_kernel(row_hbm, col_hbm, ewm_hbm, xsa_hbm, xsb_hbm, agg_out,
                col_v, row_v, ewm_v, msg_v, sg, ss, acc_sh):
    c = lax.axis_index("c")
    s = lax.axis_index("s")
    per_tile = _EPA // 16
    nch = per_tile // _CH
    nslice = _NP // 16

    def zrow(e, _):
        for q in range(_FH // 16):
            msg_v[0, e, pl.ds(q * 16, 16)] = jnp.zeros((16,), jnp.float32)
        return 0
    lax.fori_loop(0, _CH, zrow, 0)
    for j in range(nslice // _CH):
        pltpu.sync_copy(msg_v.at[0], acc_sh.at[pl.ds(s * nslice + j * _CH, _CH)])
    rem = nslice - (nslice // _CH) * _CH
    if rem:
        pltpu.sync_copy(msg_v.at[0, pl.ds(0, rem)],
                        acc_sh.at[pl.ds(s * nslice + (nslice // _CH) * _CH, rem)])
    plsc.subcore_barrier()

    def issue(idx, b):
        base = pl.multiple_of(s * per_tile + idx * _CH, 8)
        pltpu.sync_copy(row_hbm.at[pl.ds(base, _CH)],
                        row_v.at[pl.ds(b * _CH, _CH)])
        pltpu.sync_copy(col_hbm.at[pl.ds(base, _CH)], col_v.at[b])
        pltpu.sync_copy(ewm_hbm.at[pl.ds(base, _CH)],
                        ewm_v.at[pl.ds(b * _CH, _CH)])

        @pl.when(c == 0)
        def _():
            pltpu.async_copy(xsa_hbm.at[row_v.at[pl.ds(b * _CH, _CH)]],
                             msg_v.at[b], sg.at[b])

        @pl.when(c == 1)
        def _():
            pltpu.async_copy(xsb_hbm.at[row_v.at[pl.ds(b * _CH, _CH)]],
                             msg_v.at[b], sg.at[b])

    def wait_gather(b):
        @pl.when(c == 0)
        def _():
            pltpu.make_async_copy(xsa_hbm.at[row_v.at[pl.ds(b * _CH, _CH)]],
                                  msg_v.at[b], sg.at[b]).wait()

        @pl.when(c == 1)
        def _():
            pltpu.make_async_copy(xsb_hbm.at[row_v.at[pl.ds(b * _CH, _CH)]],
                                  msg_v.at[b], sg.at[b]).wait()

    def scatter(b):
        pltpu.async_copy(msg_v.at[b], acc_sh.at[col_v.at[b]], ss.at[b],
                         add=True)

    def wait_scatter(b):
        pltpu.make_async_copy(msg_v.at[b], acc_sh.at[col_v.at[b]],
                              ss.at[b]).wait()

    def scale(b):
        def sbody(g2, _):
            for u in range(4):
                e = g2 * 4 + u
                wv = plsc.load_gather(ewm_v, [jnp.full((16,), b * _CH + e,
                                                       jnp.int32)])
                for q in range(_FH // 16):
                    msg_v[b, e, pl.ds(q * 16, 16)] = (
                        msg_v[b, e, pl.ds(q * 16, 16)] * wv)
            return 0
        lax.fori_loop(0, _CH // 4, sbody, 0)

    issue(0, 0)
    issue(1, 1)

    def body(g, _):
        for b in range(3):
            i = g * 3 + b
            wait_gather(b)
            scale(b)
            scatter(b)
            nxt = (b + 2) % 3
            @pl.when(i + 2 < nch)
            def _():
                @pl.when(i >= 1)
                def _():
                    wait_scatter(nxt)
                issue(i + 2, nxt)
        return 0
    lax.fori_loop(0, nch // 3, body, 0)
    wait_scatter(0)
    wait_scatter(1)
    wait_scatter(2)
    plsc.subcore_barrier()

    nfull = nslice // _CH
    for j in range(nfull):
        sl = s * nslice + j * _CH
        pltpu.sync_copy(acc_sh.at[pl.ds(sl, _CH)],
                        agg_out.at[pl.ds(c * _NP + sl, _CH)])
    if nslice - nfull * _CH:
        sl = s * nslice + nfull * _CH
        pltpu.sync_copy(acc_sh.at[pl.ds(sl, nslice - nfull * _CH)],
                        agg_out.at[pl.ds(c * _NP + sl, nslice - nfull * _CH)])


def _tc1_body(x_ref, deg_ref, w1_ref, b1_ref, w2_ref, b2_ref, w3_ref, b3_ref,
              dis_ref, t0a_ref, t0b_ref, xsa_ref, xsb_ref):
    deg = deg_ref[0, :] + deg_ref[1, :]
    safe = jnp.where(deg > 0, deg, jnp.ones_like(deg))
    dis = jnp.where(deg > 0, lax.rsqrt(safe), jnp.zeros_like(deg))
    dis_ref[...] = dis
    dis2d = jnp.reshape(dis, (_NBLK, 1))
    w1 = w1_ref[...]
    w2 = w2_ref[...]
    w3 = w3_ref[...]
    b1 = b1_ref[...]
    b2 = b2_ref[...]
    b3 = b3_ref[...]
    for t in range(10):
        def conv(w, b):
            return (x_ref[:, t:t + 1] * w[0:1, :]
                    + x_ref[:, t + 1:t + 2] * w[1:2, :]
                    + x_ref[:, t + 2:t + 3] * w[2:3, :]) + b
        h = jax.nn.relu(conv(w1, b1) * jax.nn.sigmoid(conv(w2, b2))
                        + conv(w3, b3))
        lo = (t % 5) * 32
        if t < 5:
            t0a_ref[:, lo:lo + 32] = h
            xsa_ref[:, lo:lo + 32] = dis2d * h
        else:
            t0b_ref[:, lo:lo + 32] = h
            xsb_ref[:, lo:lo + 32] = dis2d * h


_tc1_call = pl.pallas_call(
    _tc1_body,
    grid=(_GRID,),
    in_specs=[
        pl.BlockSpec((_NBLK, 12), lambda i: (i, 0)),
        pl.BlockSpec((2, _NBLK), lambda i: (0, i)),
        pl.BlockSpec((3, 32), lambda i: (0, 0)),
        pl.BlockSpec((1, 32), lambda i: (0, 0)),
        pl.BlockSpec((3, 32), lambda i: (0, 0)),
        pl.BlockSpec((1, 32), lambda i: (0, 0)),
        pl.BlockSpec((3, 32), lambda i: (0, 0)),
        pl.BlockSpec((1, 32), lambda i: (0, 0)),
    ],
    out_specs=[
        pl.BlockSpec((_NBLK,), lambda i: (i,)),
        pl.BlockSpec((_NBLK, _FH), lambda i: (i, 0)),
        pl.BlockSpec((_NBLK, _FH), lambda i: (i, 0)),
        pl.BlockSpec((_NBLK, _FH), lambda i: (i, 0)),
        pl.BlockSpec((_NBLK, _FH), lambda i: (i, 0)),
    ],
    out_shape=[
        jax.ShapeDtypeStruct((_NP,), jnp.float32),
        jax.ShapeDtypeStruct((_NP, _FH), jnp.float32),
        jax.ShapeDtypeStruct((_NP, _FH), jnp.float32),
        jax.ShapeDtypeStruct((_NP, _FH), jnp.float32),
        jax.ShapeDtypeStruct((_NP, _FH), jnp.float32),
    ],
)


def _tc2_body(t0a_ref, t0b_ref, ag0_ref, ag1_ref, dis_ref, w0_ref, w1_ref,
              cb_ref, wa_ref, ba_ref, wb_ref, bb_ref, wc_ref, bc_ref,
              g_ref, be_ref, lw_ref, lb_ref, out_ref):
    W0 = w0_ref[...]
    W1 = w1_ref[...]
    cb = cb_ref[...]
    ndis = jnp.reshape(-dis_ref[...], (_NBLK, 1))
    G = []
    for t in range(10):
        tr = t0a_ref if t < 5 else t0b_ref
        ar = ag0_ref if t < 5 else ag1_ref
        lo = (t % 5) * 32
        xt = tr[:, lo:lo + 32]
        at = ar[:, lo:lo + 32]
        g = (jnp.dot(xt, W0, preferred_element_type=jnp.float32)
             + ndis * jnp.dot(at, W1, preferred_element_type=jnp.float32)
             + cb)
        G.append(jax.nn.relu(g))
    Wa = wa_ref[...]
    Wb = wb_ref[...]
    Wc = wc_ref[...]
    ba = ba_ref[...]
    bb = bb_ref[...]
    bc = bc_ref[...]
    Hs = []
    S = jnp.zeros((_NBLK, 32), jnp.float32)
    Q = jnp.zeros((_NBLK, 32), jnp.float32)
    for t in range(8):
        def c2(W, b):
            return (jnp.dot(G[t], W[0], preferred_element_type=jnp.float32)
                    + jnp.dot(G[t + 1], W[1], preferred_element_type=jnp.float32)
                    + jnp.dot(G[t + 2], W[2], preferred_element_type=jnp.float32)
                    + b)
        h = jax.nn.relu(c2(Wa, ba) * jax.nn.sigmoid(c2(Wb, bb)) + c2(Wc, bc))
        Hs.append(h)
        S = S + h
    m = jnp.sum(S, axis=1, keepdims=True) / 256.0
    for t in range(8):
        d = Hs[t] - m
        Q = Q + d * d
    var = jnp.sum(Q, axis=1, keepdims=True) / 256.0
    inv = lax.rsqrt(var + 1e-5)
    gam = jnp.reshape(g_ref[...], (_NBLK, 1))
    bet = jnp.reshape(be_ref[...], (_NBLK, 1))
    acc = jnp.zeros((_NBLK, 32), jnp.float32)
    for t in range(8):
        acc = acc + jax.nn.relu((Hs[t] - m) * inv * gam + bet)
    M = acc / 8.0
    out_ref[...] = (jnp.sum(M * lw_ref[...], axis=1, keepdims=True)
                    + lb_ref[...])


_tc2_call = pl.pallas_call(
    _tc2_body,
    grid=(_GRID,),
    in_specs=[
        pl.BlockSpec((_NBLK, _FH), lambda i: (i, 0)),
        pl.BlockSpec((_NBLK, _FH), lambda i: (i, 0)),
        pl.BlockSpec((_NBLK, _FH), lambda i: (i, 0)),
        pl.BlockSpec((_NBLK, _FH), lambda i: (i + _GRID, 0)),
        pl.BlockSpec((_NBLK,), lambda i: (i,)),
        pl.BlockSpec((32, 32), lambda i: (0, 0)),
        pl.BlockSpec((32, 32), lambda i: (0, 0)),
        pl.BlockSpec((1, 32), lambda i: (0, 0)),
        pl.BlockSpec((3, 32, 32), lambda i: (0, 0, 0)),
        pl.BlockSpec((1, 32), lambda i: (0, 0)),
        pl.BlockSpec((3, 32, 32), lambda i: (0, 0, 0)),
        pl.BlockSpec((1, 32), lambda i: (0, 0)),
        pl.BlockSpec((3, 32, 32), lambda i: (0, 0, 0)),
        pl.BlockSpec((1, 32), lambda i: (0, 0)),
        pl.BlockSpec((_NBLK,), lambda i: (i,)),
        pl.BlockSpec((_NBLK,), lambda i: (i,)),
        pl.BlockSpec((1, 32), lambda i: (0, 0)),
        pl.BlockSpec((1, 1), lambda i: (0, 0)),
    ],
    out_specs=pl.BlockSpec((_NBLK, 1), lambda i: (i, 0)),
    out_shape=jax.ShapeDtypeStruct((_NP, 1), jnp.float32),
)


def kernel(x, edge_index, edge_weight, tc1_w1, tc1_b1, tc1_w2, tc1_b2,
           tc1_w3, tc1_b3, cheb_w0, cheb_w1, cheb_b, tc2_w1, tc2_b1,
           tc2_w2, tc2_b2, tc2_w3, tc2_b3, bn_gamma, bn_beta, lin_w, lin_b):
    row, col = edge_index[0], edge_index[1]

    ewm = _tc0_call(row.reshape(5000, 128), col.reshape(5000, 128),
                    edge_weight.reshape(5000, 128)).reshape(-1)
    padn = _EPD - _E
    rowp = jnp.concatenate([row, jnp.arange(padn, dtype=jnp.int32) % _N])
    ewmp = jnp.concatenate([ewm, jnp.zeros((padn,), jnp.float32)])
    deg2 = _deg_kernel(rowp, ewmp).reshape(2, _NP)

    x2 = jnp.pad(jnp.transpose(x[0, :, :, 0]), ((0, _NP - _N), (0, 0)))
    w1 = tc1_w1[:, 0, 0, :].T
    w2 = tc1_w2[:, 0, 0, :].T
    w3 = tc1_w3[:, 0, 0, :].T
    dis, t0a, t0b, xsa, xsb = _tc1_call(x2, deg2, w1, tc1_b1[None, :], w2,
                                        tc1_b2[None, :], w3, tc1_b3[None, :])

    pada = _EPA - _E
    rowa = jnp.concatenate([row, jnp.arange(pada, dtype=jnp.int32) % _N])
    cola = jnp.concatenate([col, jnp.arange(pada, dtype=jnp.int32) % _N])
    ewma = jnp.concatenate([ewm, jnp.zeros((pada,), jnp.float32)])
    agg = _agg_kernel(rowa, cola, ewma, xsa, xsb)

    wa = jnp.transpose(tc2_w1, (3, 1, 2, 0))[:, :, 0, :]
    wb = jnp.transpose(tc2_w2, (3, 1, 2, 0))[:, :, 0, :]
    wc = jnp.transpose(tc2_w3, (3, 1, 2, 0))[:, :, 0, :]
    gam = jnp.pad(bn_gamma, (0, _NP - _N))
    bet = jnp.pad(bn_beta, (0, _NP - _N))
    out = _tc2_call(t0a, t0b, agg, agg, dis, cheb_w0, cheb_w1,
                    cheb_b[None, :], wa, tc2_b1[None, :], wb, tc2_b2[None, :],
                    wc, tc2_b3[None, :], gam, bet, lin_w, lin_b[None, :])
    return out[:_N]


# speedup vs baseline: 1.5096x; 1.5096x over previous
import functools

import jax
import jax.numpy as jnp
from jax import lax
from jax.experimental import pallas as pl
from jax.experimental.pallas import tpu as pltpu
from jax.experimental.pallas import tpu_sc as plsc

_N = 10000
_NP = 10240
_E = 640000
_CH = 56
_EPA = 642432
_FH = 160
_NBLK = 1024
_GRID = _NP // _NBLK

_mesh = plsc.VectorSubcoreMesh(core_axis_name="c", subcore_axis_name="s")
_sc_params = pltpu.CompilerParams(needs_layout_passes=False,
                                  use_tc_tiling_on_sc=False)


_DCH = 128
_EPD = 647168


@functools.partial(
    pl.kernel,
    out_type=jax.ShapeDtypeStruct((2 * _NP,), jnp.float32),
    mesh=_mesh,
    scratch_types=[
        pltpu.VMEM((2, _DCH), jnp.int32),
        pltpu.VMEM((2, _DCH), jnp.float32),
        pltpu.VMEM((_NP // 16,), jnp.float32),
        pltpu.SemaphoreType.DMA((2,)),
        pltpu.SemaphoreType.DMA((2,)),
        pltpu.VMEM_SHARED((_NP,), jnp.float32),
    ],
    compiler_params=_sc_params,
)
def _deg_kernel(row_hbm, ewm_hbm, deg_out,
                row_v, upd_v, z_v, slm, sd, deg_sh):
    c = lax.axis_index("c")
    s = lax.axis_index("s")
    nslice = _NP // 16

    def zb(j, _):
        z_v[pl.ds(j * 16, 16)] = jnp.zeros((16,), jnp.float32)
        return 0
    lax.fori_loop(0, nslice // 16, zb, 0)
    pltpu.sync_copy(z_v, deg_sh.at[pl.ds(s * nslice, nslice)])
    plsc.subcore_barrier()

    per_core = _EPD // 2
    per_tile = per_core // 16
    nch = per_tile // _DCH

    def issue_load(idx, b):
        base = pl.multiple_of(c * per_core + s * per_tile + idx * _DCH, 8)
        pltpu.async_copy(row_hbm.at[pl.ds(base, _DCH)], row_v.at[b],
                         slm.at[b])
        pltpu.async_copy(ewm_hbm.at[pl.ds(base, _DCH)], upd_v.at[b],
                         slm.at[b])

    def wait_load(idx, b):
        base = pl.multiple_of(c * per_core + s * per_tile + idx * _DCH, 8)
        pltpu.make_async_copy(row_hbm.at[pl.ds(base, _DCH)], row_v.at[b],
                              slm.at[b]).wait()
        pltpu.make_async_copy(ewm_hbm.at[pl.ds(base, _DCH)], upd_v.at[b],
                              slm.at[b]).wait()

    def scatter(b):
        pltpu.async_copy(upd_v.at[b], deg_sh.at[row_v.at[b]], sd.at[b],
                         add=True)

    def wait_scatter(b):
        pltpu.make_async_copy(upd_v.at[b], deg_sh.at[row_v.at[b]],
                              sd.at[b]).wait()

    issue_load(0, 0)

    def body(g, _):
        i0 = g * 2

        @pl.when(g >= 1)
        def _():
            wait_scatter(1)
        issue_load(i0 + 1, 1)
        wait_load(i0, 0)
        scatter(0)

        @pl.when(i0 + 2 < nch)
        def _():
            wait_scatter(0)
            issue_load(i0 + 2, 0)
        wait_load(i0 + 1, 1)
        scatter(1)
        return 0
    lax.fori_loop(0, nch // 2, body, 0)
    wait_scatter(0)
    wait_scatter(1)
    plsc.subcore_barrier()
    pltpu.sync_copy(deg_sh.at[pl.ds(s * nslice, nslice)],
                    deg_out.at[pl.ds(c * _NP + s * nslice, nslice)])


def _tc0_body(r_ref, c_ref, w_ref, o_ref):
    o_ref[...] = jnp.where(r_ref[...] == c_ref[...],
                           jnp.zeros_like(w_ref[...]), w_ref[...])


_tc0_call = pl.pallas_call(
    _tc0_body,
    grid=(5,),
    in_specs=[
        pl.BlockSpec((1000, 128), lambda i: (i, 0)),
        pl.BlockSpec((1000, 128), lambda i: (i, 0)),
        pl.BlockSpec((1000, 128), lambda i: (i, 0)),
    ],
    out_specs=pl.BlockSpec((1000, 128), lambda i: (i, 0)),
    out_shape=jax.ShapeDtypeStruct((5000, 128), jnp.float32),
)


@functools.partial(
    pl.kernel,
    out_type=jax.ShapeDtypeStruct((2 * _NP, _FH), jnp.float32),
    mesh=_mesh,
    scratch_types=[
        pltpu.VMEM((3, _CH), jnp.int32),
        pltpu.VMEM((3 * _CH,), jnp.int32),
        pltpu.VMEM((3 * _CH,), jnp.float32),
        pltpu.VMEM((3, _CH, _FH), jnp.float32),
        pltpu.SemaphoreType.DMA((3,)),
        pltpu.SemaphoreType.DMA((3,)),
        pltpu.SemaphoreType.DMA((3,)),
        pltpu.VMEM_SHARED((_NP, _FH), jnp.float32),
    ],
    compiler_params=_sc_params,
)
def _agg_kernel(row_hbm, col_hbm, ewm_hbm, xsa_hbm, xsb_hbm, agg_out,
                col_v, row_v, ewm_v, msg_v, sg, ss, sl, acc_sh):
    c = lax.axis_index("c")
    s = lax.axis_index("s")
    per_tile = _EPA // 16
    nch = per_tile // _CH
    nslice = _NP // 16

    def zrow(e, _):
        for q in range(_FH // 16):
            msg_v[0, e, pl.ds(q * 16, 16)] = jnp.zeros((16,), jnp.float32)
        return 0
    lax.fori_loop(0, _CH, zrow, 0)
    for j in range(nslice // _CH):
        pltpu.sync_copy(msg_v.at[0], acc_sh.at[pl.ds(s * nslice + j * _CH, _CH)])
    rem = nslice - (nslice // _CH) * _CH
    if rem:
        pltpu.sync_copy(msg_v.at[0, pl.ds(0, rem)],
                        acc_sh.at[pl.ds(s * nslice + (nslice // _CH) * _CH, rem)])
    plsc.subcore_barrier()

    def issue_loads(idx, b):
        base = pl.multiple_of(s * per_tile + idx * _CH, 8)
        pltpu.async_copy(row_hbm.at[pl.ds(base, _CH)],
                         row_v.at[pl.ds(b * _CH, _CH)], sl.at[b])
        pltpu.async_copy(col_hbm.at[pl.ds(base, _CH)], col_v.at[b], sl.at[b])
        pltpu.async_copy(ewm_hbm.at[pl.ds(base, _CH)],
                         ewm_v.at[pl.ds(b * _CH, _CH)], sl.at[b])

    def wait_loads(idx, b):
        base = pl.multiple_of(s * per_tile + idx * _CH, 8)
        pltpu.make_async_copy(row_hbm.at[pl.ds(base, _CH)],
                              row_v.at[pl.ds(b * _CH, _CH)], sl.at[b]).wait()
        pltpu.make_async_copy(col_hbm.at[pl.ds(base, _CH)], col_v.at[b],
                              sl.at[b]).wait()
        pltpu.make_async_copy(ewm_hbm.at[pl.ds(base, _CH)],
                              ewm_v.at[pl.ds(b * _CH, _CH)], sl.at[b]).wait()

    def issue_gather(b):
        @pl.when(c == 0)
        def _():
            pltpu.async_copy(xsa_hbm.at[row_v.at[pl.ds(b * _CH, _CH)]],
                             msg_v.at[b], sg.at[b])

        @pl.when(c == 1)
        def _():
            pltpu.async_copy(xsb_hbm.at[row_v.at[pl.ds(b * _CH, _CH)]],
                             msg_v.at[b], sg.at[b])

    def wait_gather(b):
        @pl.when(c == 0)
        def _():
            pltpu.make_async_copy(xsa_hbm.at[row_v.at[pl.ds(b * _CH, _CH)]],
                                  msg_v.at[b], sg.at[b]).wait()

        @pl.when(c == 1)
        def _():
            pltpu.make_async_copy(xsb_hbm.at[row_v.at[pl.ds(b * _CH, _CH)]],
                                  msg_v.at[b], sg.at[b]).wait()

    def scatter(b):
        pltpu.async_copy(msg_v.at[b], acc_sh.at[col_v.at[b]], ss.at[b],
                         add=True)

    def wait_scatter(b):
        pltpu.make_async_copy(msg_v.at[b], acc_sh.at[col_v.at[b]],
                              ss.at[b]).wait()

    def scale(b):
        def sbody(g2, _):
            for u in range(4):
                e = g2 * 4 + u
                wv = plsc.load_gather(ewm_v, [jnp.full((16,), b * _CH + e,
                                                       jnp.int32)])
                for q in range(_FH // 16):
                    msg_v[b, e, pl.ds(q * 16, 16)] = (
                        msg_v[b, e, pl.ds(q * 16, 16)] * wv)
            return 0
        lax.fori_loop(0, _CH // 4, sbody, 0)

    issue_loads(0, 0)
    wait_loads(0, 0)
    issue_gather(0)
    issue_loads(1, 1)

    def body(g, _):
        for b in range(3):
            i = g * 3 + b
            wait_gather(b)
            b1 = (b + 1) % 3

            @pl.when(i + 1 < nch)
            def _():
                wait_loads(i + 1, b1)
                issue_gather(b1)
            scale(b)
            scatter(b)
            nxt = (b + 2) % 3

            @pl.when(i + 2 < nch)
            def _():
                @pl.when(i >= 1)
                def _():
                    wait_scatter(nxt)
                issue_loads(i + 2, nxt)
        return 0
    lax.fori_loop(0, nch // 3, body, 0)
    wait_scatter(0)
    wait_scatter(1)
    wait_scatter(2)
    plsc.subcore_barrier()

    nfull = nslice // _CH
    for j in range(nfull):
        sl = s * nslice + j * _CH
        pltpu.sync_copy(acc_sh.at[pl.ds(sl, _CH)],
                        agg_out.at[pl.ds(c * _NP + sl, _CH)])
    if nslice - nfull * _CH:
        sl = s * nslice + nfull * _CH
        pltpu.sync_copy(acc_sh.at[pl.ds(sl, nslice - nfull * _CH)],
                        agg_out.at[pl.ds(c * _NP + sl, nslice - nfull * _CH)])


def _tc1_body(x_ref, deg_ref, w1_ref, b1_ref, w2_ref, b2_ref, w3_ref, b3_ref,
              dis_ref, t0a_ref, t0b_ref, xsa_ref, xsb_ref):
    deg = deg_ref[0, :] + deg_ref[1, :]
    safe = jnp.where(deg > 0, deg, jnp.ones_like(deg))
    dis = jnp.where(deg > 0, lax.rsqrt(safe), jnp.zeros_like(deg))
    dis_ref[...] = dis
    dis2d = jnp.reshape(dis, (_NBLK, 1))
    w1 = w1_ref[...]
    w2 = w2_ref[...]
    w3 = w3_ref[...]
    b1 = b1_ref[...]
    b2 = b2_ref[...]
    b3 = b3_ref[...]
    for t in range(10):
        def conv(w, b):
            return (x_ref[:, t:t + 1] * w[0:1, :]
                    + x_ref[:, t + 1:t + 2] * w[1:2, :]
                    + x_ref[:, t + 2:t + 3] * w[2:3, :]) + b
        h = jax.nn.relu(conv(w1, b1) * jax.nn.sigmoid(conv(w2, b2))
                        + conv(w3, b3))
        lo = (t % 5) * 32
        if t < 5:
            t0a_ref[:, lo:lo + 32] = h
            xsa_ref[:, lo:lo + 32] = dis2d * h
        else:
            t0b_ref[:, lo:lo + 32] = h
            xsb_ref[:, lo:lo + 32] = dis2d * h


_tc1_call = pl.pallas_call(
    _tc1_body,
    grid=(_GRID,),
    in_specs=[
        pl.BlockSpec((_NBLK, 12), lambda i: (i, 0)),
        pl.BlockSpec((2, _NBLK), lambda i: (0, i)),
        pl.BlockSpec((3, 32), lambda i: (0, 0)),
        pl.BlockSpec((1, 32), lambda i: (0, 0)),
        pl.BlockSpec((3, 32), lambda i: (0, 0)),
        pl.BlockSpec((1, 32), lambda i: (0, 0)),
        pl.BlockSpec((3, 32), lambda i: (0, 0)),
        pl.BlockSpec((1, 32), lambda i: (0, 0)),
    ],
    out_specs=[
        pl.BlockSpec((_NBLK,), lambda i: (i,)),
        pl.BlockSpec((_NBLK, _FH), lambda i: (i, 0)),
        pl.BlockSpec((_NBLK, _FH), lambda i: (i, 0)),
        pl.BlockSpec((_NBLK, _FH), lambda i: (i, 0)),
        pl.BlockSpec((_NBLK, _FH), lambda i: (i, 0)),
    ],
    out_shape=[
        jax.ShapeDtypeStruct((_NP,), jnp.float32),
        jax.ShapeDtypeStruct((_NP, _FH), jnp.float32),
        jax.ShapeDtypeStruct((_NP, _FH), jnp.float32),
        jax.ShapeDtypeStruct((_NP, _FH), jnp.float32),
        jax.ShapeDtypeStruct((_NP, _FH), jnp.float32),
    ],
)


def _tc2_body(t0a_ref, t0b_ref, ag0_ref, ag1_ref, dis_ref, w0_ref, w1_ref,
              cb_ref, wa_ref, ba_ref, wb_ref, bb_ref, wc_ref, bc_ref,
              g_ref, be_ref, lw_ref, lb_ref, out_ref):
    W0 = w0_ref[...]
    W1 = w1_ref[...]
    cb = cb_ref[...]
    ndis = jnp.reshape(-dis_ref[...], (_NBLK, 1))
    G = []
    for t in range(10):
        tr = t0a_ref if t < 5 else t0b_ref
        ar = ag0_ref if t < 5 else ag1_ref
        lo = (t % 5) * 32
        xt = tr[:, lo:lo + 32]
        at = ar[:, lo:lo + 32]
        g = (jnp.dot(xt, W0, preferred_element_type=jnp.float32)
             + ndis * jnp.dot(at, W1, preferred_element_type=jnp.float32)
             + cb)
        G.append(jax.nn.relu(g))
    Wa = wa_ref[...]
    Wb = wb_ref[...]
    Wc = wc_ref[...]
    ba = ba_ref[...]
    bb = bb_ref[...]
    bc = bc_ref[...]
    Hs = []
    S = jnp.zeros((_NBLK, 32), jnp.float32)
    Q = jnp.zeros((_NBLK, 32), jnp.float32)
    for t in range(8):
        def c2(W, b):
            return (jnp.dot(G[t], W[0], preferred_element_type=jnp.float32)
                    + jnp.dot(G[t + 1], W[1], preferred_element_type=jnp.float32)
                    + jnp.dot(G[t + 2], W[2], preferred_element_type=jnp.float32)
                    + b)
        h = jax.nn.relu(c2(Wa, ba) * jax.nn.sigmoid(c2(Wb, bb)) + c2(Wc, bc))
        Hs.append(h)
        S = S + h
    m = jnp.sum(S, axis=1, keepdims=True) / 256.0
    for t in range(8):
        d = Hs[t] - m
        Q = Q + d * d
    var = jnp.sum(Q, axis=1, keepdims=True) / 256.0
    inv = lax.rsqrt(var + 1e-5)
    gam = jnp.reshape(g_ref[...], (_NBLK, 1))
    bet = jnp.reshape(be_ref[...], (_NBLK, 1))
    acc = jnp.zeros((_NBLK, 32), jnp.float32)
    for t in range(8):
        acc = acc + jax.nn.relu((Hs[t] - m) * inv * gam + bet)
    M = acc / 8.0
    out_ref[...] = (jnp.sum(M * lw_ref[...], axis=1, keepdims=True)
                    + lb_ref[...])


_tc2_call = pl.pallas_call(
    _tc2_body,
    grid=(_GRID,),
    in_specs=[
        pl.BlockSpec((_NBLK, _FH), lambda i: (i, 0)),
        pl.BlockSpec((_NBLK, _FH), lambda i: (i, 0)),
        pl.BlockSpec((_NBLK, _FH), lambda i: (i, 0)),
        pl.BlockSpec((_NBLK, _FH), lambda i: (i + _GRID, 0)),
        pl.BlockSpec((_NBLK,), lambda i: (i,)),
        pl.BlockSpec((32, 32), lambda i: (0, 0)),
        pl.BlockSpec((32, 32), lambda i: (0, 0)),
        pl.BlockSpec((1, 32), lambda i: (0, 0)),
        pl.BlockSpec((3, 32, 32), lambda i: (0, 0, 0)),
        pl.BlockSpec((1, 32), lambda i: (0, 0)),
        pl.BlockSpec((3, 32, 32), lambda i: (0, 0, 0)),
        pl.BlockSpec((1, 32), lambda i: (0, 0)),
        pl.BlockSpec((3, 32, 32), lambda i: (0, 0, 0)),
        pl.BlockSpec((1, 32), lambda i: (0, 0)),
        pl.BlockSpec((_NBLK,), lambda i: (i,)),
        pl.BlockSpec((_NBLK,), lambda i: (i,)),
        pl.BlockSpec((1, 32), lambda i: (0, 0)),
        pl.BlockSpec((1, 1), lambda i: (0, 0)),
    ],
    out_specs=pl.BlockSpec((_NBLK, 1), lambda i: (i, 0)),
    out_shape=jax.ShapeDtypeStruct((_NP, 1), jnp.float32),
)


def kernel(x, edge_index, edge_weight, tc1_w1, tc1_b1, tc1_w2, tc1_b2,
           tc1_w3, tc1_b3, cheb_w0, cheb_w1, cheb_b, tc2_w1, tc2_b1,
           tc2_w2, tc2_b2, tc2_w3, tc2_b3, bn_gamma, bn_beta, lin_w, lin_b):
    row, col = edge_index[0], edge_index[1]

    ewm = _tc0_call(row.reshape(5000, 128), col.reshape(5000, 128),
                    edge_weight.reshape(5000, 128)).reshape(-1)
    padn = _EPD - _E
    rowp = jnp.concatenate([row, jnp.arange(padn, dtype=jnp.int32) % _N])
    ewmp = jnp.concatenate([ewm, jnp.zeros((padn,), jnp.float32)])
    deg2 = _deg_kernel(rowp, ewmp).reshape(2, _NP)

    x2 = jnp.pad(jnp.transpose(x[0, :, :, 0]), ((0, _NP - _N), (0, 0)))
    w1 = tc1_w1[:, 0, 0, :].T
    w2 = tc1_w2[:, 0, 0, :].T
    w3 = tc1_w3[:, 0, 0, :].T
    dis, t0a, t0b, xsa, xsb = _tc1_call(x2, deg2, w1, tc1_b1[None, :], w2,
                                        tc1_b2[None, :], w3, tc1_b3[None, :])

    pada = _EPA - _E
    rowa = jnp.concatenate([row, jnp.arange(pada, dtype=jnp.int32) % _N])
    cola = jnp.concatenate([col, jnp.arange(pada, dtype=jnp.int32) % _N])
    ewma = jnp.concatenate([ewm, jnp.zeros((pada,), jnp.float32)])
    agg = _agg_kernel(rowa, cola, ewma, xsa, xsb)

    wa = jnp.transpose(tc2_w1, (3, 1, 2, 0))[:, :, 0, :]
    wb = jnp.transpose(tc2_w2, (3, 1, 2, 0))[:, :, 0, :]
    wc = jnp.transpose(tc2_w3, (3, 1, 2, 0))[:, :, 0, :]
    gam = jnp.pad(bn_gamma, (0, _NP - _N))
    bet = jnp.pad(bn_beta, (0, _NP - _N))
    out = _tc2_call(t0a, t0b, agg, agg, dis, cheb_w0, cheb_w1,
                    cheb_b[None, :], wa, tc2_b1[None, :], wb, tc2_b2[None, :],
                    wc, tc2_b3[None, :], gam, bet, lin_w, lin_b[None, :])
    return out[:_N]
